# baseline (device time: 33641 ns/iter reference)
import jax
import jax.numpy as jnp
from jax import lax
from jax.experimental import pallas as pl
from jax.experimental.pallas import tpu as pltpu

M = 2048
HALF = 1024
D = 1024


def kernel(partial, gamma):
    partial2 = partial.reshape(M, D)
    gamma2 = gamma.reshape(1, D)

    def body(p_ref, g_ref, o_ref, send_buf, recv_buf, send_sem, recv_sem):
        my_x = lax.axis_index("x")
        my_y = lax.axis_index("y")
        my_z = lax.axis_index("z")
        nbr = (1 - my_x, my_y, my_z)

        barrier = pltpu.get_barrier_semaphore()
        pl.semaphore_signal(
            barrier, inc=1, device_id=nbr, device_id_type=pl.DeviceIdType.MESH
        )
        pl.semaphore_wait(barrier, 1)

        nbr_base = (1 - my_x) * HALF
        send_buf[:, :] = p_ref[pl.ds(nbr_base, HALF), :].astype(jnp.bfloat16)

        rdma = pltpu.make_async_remote_copy(
            src_ref=send_buf,
            dst_ref=recv_buf,
            send_sem=send_sem,
            recv_sem=recv_sem,
            device_id=nbr,
            device_id_type=pl.DeviceIdType.MESH,
        )
        rdma.start()
        rdma.wait()

        my_base = my_x * HALF
        s = recv_buf[:, :].astype(jnp.float32) + p_ref[pl.ds(my_base, HALF), :]
        ms = jnp.mean(s * s, axis=1, keepdims=True) + 1e-6
        o_ref[:, :] = s * lax.rsqrt(ms) * g_ref[:, :]

    return pl.pallas_call(
        body,
        out_shape=jax.ShapeDtypeStruct((HALF, D), jnp.float32),
        in_specs=[
            pl.BlockSpec(memory_space=pltpu.VMEM),
            pl.BlockSpec(memory_space=pltpu.VMEM),
        ],
        out_specs=pl.BlockSpec(memory_space=pltpu.VMEM),
        scratch_shapes=[
            pltpu.VMEM((HALF, D), jnp.bfloat16),
            pltpu.VMEM((HALF, D), jnp.bfloat16),
            pltpu.SemaphoreType.DMA,
            pltpu.SemaphoreType.DMA,
        ],
        compiler_params=pltpu.CompilerParams(collective_id=0),
    )(partial2, gamma2)


# device time: 27570 ns/iter; 1.2202x vs baseline; 1.2202x over previous
import jax
import jax.numpy as jnp
from jax import lax
from jax.experimental import pallas as pl
from jax.experimental.pallas import tpu as pltpu

M = 2048
HALF = 1024
ROWS = 512
D = 1024
C = 4
CH = ROWS // C


def kernel(partial, gamma):
    partial2 = partial.reshape(M, D)
    gamma2 = gamma.reshape(1, D)

    def body(
        p_ref, g_ref, o_ref,
        sx, rx, sz, rz,
        sx_sem, rx_sem, sz_sem, rz_sem,
    ):
        my_x = lax.axis_index("x")
        my_y = lax.axis_index("y")
        my_z = lax.axis_index("z")
        xn = (1 - my_x, my_y, my_z)
        zn = (my_x, my_y, 1 - my_z)

        barrier = pltpu.get_barrier_semaphore()
        for nbr in (xn, zn):
            pl.semaphore_signal(
                barrier, inc=1, device_id=nbr,
                device_id_type=pl.DeviceIdType.MESH,
            )
        pl.semaphore_wait(barrier, 2)

        sx[:, :] = p_ref[
            pl.ds((1 - my_x) * HALF + my_z * ROWS, ROWS), :
        ].astype(jnp.bfloat16)

        x_rdmas = []
        for c in range(C):
            r = pltpu.make_async_remote_copy(
                src_ref=sx.at[pl.ds(c * CH, CH)],
                dst_ref=rx.at[pl.ds(c * CH, CH)],
                send_sem=sx_sem.at[c],
                recv_sem=rx_sem.at[c],
                device_id=xn,
                device_id_type=pl.DeviceIdType.MESH,
            )
            r.start()
            x_rdmas.append(r)

        my_base = my_x * HALF + my_z * ROWS
        z_rdmas = []
        for c in range(C):
            x_rdmas[c].wait_recv()
            s = rx[pl.ds(c * CH, CH), :].astype(jnp.float32) + p_ref[
                pl.ds(my_base + c * CH, CH), :
            ]
            o_ref[pl.ds(my_z * ROWS + c * CH, CH), :] = s
            sz[pl.ds(c * CH, CH), :] = s.astype(jnp.bfloat16)
            r = pltpu.make_async_remote_copy(
                src_ref=sz.at[pl.ds(c * CH, CH)],
                dst_ref=rz.at[pl.ds(c * CH, CH)],
                send_sem=sz_sem.at[c],
                recv_sem=rz_sem.at[c],
                device_id=zn,
                device_id_type=pl.DeviceIdType.MESH,
            )
            r.start()
            z_rdmas.append(r)

        other_base = (1 - my_z) * ROWS
        for c in range(C):
            z_rdmas[c].wait_recv()
            o_ref[pl.ds(other_base + c * CH, CH), :] = rz[
                pl.ds(c * CH, CH), :
            ].astype(jnp.float32)

        s = o_ref[:, :]
        ms = jnp.mean(s * s, axis=1, keepdims=True) + 1e-6
        o_ref[:, :] = s * lax.rsqrt(ms) * g_ref[:, :]

        for c in range(C):
            x_rdmas[c].wait_send()
            z_rdmas[c].wait_send()

    return pl.pallas_call(
        body,
        out_shape=jax.ShapeDtypeStruct((HALF, D), jnp.float32),
        in_specs=[
            pl.BlockSpec(memory_space=pltpu.VMEM),
            pl.BlockSpec(memory_space=pltpu.VMEM),
        ],
        out_specs=pl.BlockSpec(memory_space=pltpu.VMEM),
        scratch_shapes=[
            pltpu.VMEM((ROWS, D), jnp.bfloat16),
            pltpu.VMEM((ROWS, D), jnp.bfloat16),
            pltpu.VMEM((ROWS, D), jnp.bfloat16),
            pltpu.VMEM((ROWS, D), jnp.bfloat16),
            pltpu.SemaphoreType.DMA((C,)),
            pltpu.SemaphoreType.DMA((C,)),
            pltpu.SemaphoreType.DMA((C,)),
            pltpu.SemaphoreType.DMA((C,)),
        ],
        compiler_params=pltpu.CompilerParams(collective_id=0),
    )(partial2, gamma2)


# device time: 26928 ns/iter; 1.2493x vs baseline; 1.0238x over previous
import jax
import jax.numpy as jnp
from jax import lax
from jax.experimental import pallas as pl
from jax.experimental.pallas import tpu as pltpu

M = 2048
HALF = 1024
ROWS = 512
D = 1024
C = 4
CH = ROWS // C


def kernel(partial, gamma):
    partial2 = partial.reshape(M, D)
    gamma2 = gamma.reshape(1, D)

    def body(
        p_ref, g_ref, o_ref,
        sx, rx, sz, rz,
        sx_sem, rx_sem, sz_sem, rz_sem,
    ):
        my_x = lax.axis_index("x")
        my_y = lax.axis_index("y")
        my_z = lax.axis_index("z")
        xn = (1 - my_x, my_y, my_z)
        zn = (my_x, my_y, 1 - my_z)

        barrier = pltpu.get_barrier_semaphore()
        for nbr in (xn, zn):
            pl.semaphore_signal(
                barrier, inc=1, device_id=nbr,
                device_id_type=pl.DeviceIdType.MESH,
            )
        pl.semaphore_wait(barrier, 2)

        send_base = (1 - my_x) * HALF + my_z * ROWS
        x_rdmas = []
        for c in range(C):
            sx[pl.ds(c * CH, CH), :] = p_ref[
                pl.ds(send_base + c * CH, CH), :
            ].astype(jnp.bfloat16)
            r = pltpu.make_async_remote_copy(
                src_ref=sx.at[pl.ds(c * CH, CH)],
                dst_ref=rx.at[pl.ds(c * CH, CH)],
                send_sem=sx_sem.at[c],
                recv_sem=rx_sem.at[c],
                device_id=xn,
                device_id_type=pl.DeviceIdType.MESH,
            )
            r.start()
            x_rdmas.append(r)

        my_base = my_x * HALF + my_z * ROWS
        z_rdmas = []
        for c in range(C):
            x_rdmas[c].wait_recv()
            s = rx[pl.ds(c * CH, CH), :].astype(jnp.float32) + p_ref[
                pl.ds(my_base + c * CH, CH), :
            ]
            sz[pl.ds(c * CH, CH), :] = s.astype(jnp.bfloat16)
            r = pltpu.make_async_remote_copy(
                src_ref=sz.at[pl.ds(c * CH, CH)],
                dst_ref=rz.at[pl.ds(c * CH, CH)],
                send_sem=sz_sem.at[c],
                recv_sem=rz_sem.at[c],
                device_id=zn,
                device_id_type=pl.DeviceIdType.MESH,
            )
            r.start()
            z_rdmas.append(r)
            ms = jnp.mean(s * s, axis=1, keepdims=True) + 1e-6
            o_ref[pl.ds(my_z * ROWS + c * CH, CH), :] = (
                s * lax.rsqrt(ms) * g_ref[:, :]
            )

        other_base = (1 - my_z) * ROWS
        for c in range(C):
            z_rdmas[c].wait_recv()
            v = rz[pl.ds(c * CH, CH), :].astype(jnp.float32)
            ms = jnp.mean(v * v, axis=1, keepdims=True) + 1e-6
            o_ref[pl.ds(other_base + c * CH, CH), :] = (
                v * lax.rsqrt(ms) * g_ref[:, :]
            )

        for c in range(C):
            x_rdmas[c].wait_send()
            z_rdmas[c].wait_send()

    return pl.pallas_call(
        body,
        out_shape=jax.ShapeDtypeStruct((HALF, D), jnp.float32),
        in_specs=[
            pl.BlockSpec(memory_space=pltpu.VMEM),
            pl.BlockSpec(memory_space=pltpu.VMEM),
        ],
        out_specs=pl.BlockSpec(memory_space=pltpu.VMEM),
        scratch_shapes=[
            pltpu.VMEM((ROWS, D), jnp.bfloat16),
            pltpu.VMEM((ROWS, D), jnp.bfloat16),
            pltpu.VMEM((ROWS, D), jnp.bfloat16),
            pltpu.VMEM((ROWS, D), jnp.bfloat16),
            pltpu.SemaphoreType.DMA((C,)),
            pltpu.SemaphoreType.DMA((C,)),
            pltpu.SemaphoreType.DMA((C,)),
            pltpu.SemaphoreType.DMA((C,)),
        ],
        compiler_params=pltpu.CompilerParams(collective_id=0),
    )(partial2, gamma2)


# device time: 25559 ns/iter; 1.3162x vs baseline; 1.0536x over previous
import jax
import jax.numpy as jnp
from jax import lax
from jax.experimental import pallas as pl
from jax.experimental.pallas import tpu as pltpu

M = 2048
HALF = 1024
ROWS = 512
D = 1024
C = 8
CH = ROWS // C


def kernel(partial, gamma):
    gamma2 = gamma.reshape(1, D)

    def body(
        p_ref, g_ref, o_ref,
        sx, rx, sz, rz,
        sx_sem, rx_sem, sz_sem, rz_sem,
    ):
        my_x = lax.axis_index("x")
        my_y = lax.axis_index("y")
        my_z = lax.axis_index("z")
        xn = (1 - my_x, my_y, my_z)
        zn = (my_x, my_y, 1 - my_z)

        barrier = pltpu.get_barrier_semaphore()
        for nbr in (xn, zn):
            pl.semaphore_signal(
                barrier, inc=1, device_id=nbr,
                device_id_type=pl.DeviceIdType.MESH,
            )
        pl.semaphore_wait(barrier, 2)

        send_base = (1 - my_x) * HALF + my_z * ROWS
        x_rdmas = []
        for c in range(C):
            sx[pl.ds(c * CH, CH), :] = p_ref[
                0, pl.ds(send_base + c * CH, CH), :
            ].astype(jnp.bfloat16)
            r = pltpu.make_async_remote_copy(
                src_ref=sx.at[pl.ds(c * CH, CH)],
                dst_ref=rx.at[pl.ds(c * CH, CH)],
                send_sem=sx_sem.at[c],
                recv_sem=rx_sem.at[c],
                device_id=xn,
                device_id_type=pl.DeviceIdType.MESH,
            )
            r.start()
            x_rdmas.append(r)

        my_base = my_x * HALF + my_z * ROWS
        z_rdmas = []
        for c in range(C):
            x_rdmas[c].wait_recv()
            s = rx[pl.ds(c * CH, CH), :].astype(jnp.float32) + p_ref[
                0, pl.ds(my_base + c * CH, CH), :
            ]
            sz[pl.ds(c * CH, CH), :] = s.astype(jnp.bfloat16)
            r = pltpu.make_async_remote_copy(
                src_ref=sz.at[pl.ds(c * CH, CH)],
                dst_ref=rz.at[pl.ds(c * CH, CH)],
                send_sem=sz_sem.at[c],
                recv_sem=rz_sem.at[c],
                device_id=zn,
                device_id_type=pl.DeviceIdType.MESH,
            )
            r.start()
            z_rdmas.append(r)
            ms = jnp.mean(s * s, axis=1, keepdims=True) + 1e-6
            o_ref[pl.ds(my_z * ROWS + c * CH, CH), :] = (
                s * lax.rsqrt(ms) * g_ref[:, :]
            )

        other_base = (1 - my_z) * ROWS
        for c in range(C):
            z_rdmas[c].wait_recv()
            v = rz[pl.ds(c * CH, CH), :].astype(jnp.float32)
            ms = jnp.mean(v * v, axis=1, keepdims=True) + 1e-6
            o_ref[pl.ds(other_base + c * CH, CH), :] = (
                v * lax.rsqrt(ms) * g_ref[:, :]
            )

        for c in range(C):
            x_rdmas[c].wait_send()
            z_rdmas[c].wait_send()

    return pl.pallas_call(
        body,
        out_shape=jax.ShapeDtypeStruct((HALF, D), jnp.float32),
        in_specs=[
            pl.BlockSpec(memory_space=pltpu.VMEM),
            pl.BlockSpec(memory_space=pltpu.VMEM),
        ],
        out_specs=pl.BlockSpec(memory_space=pltpu.VMEM),
        scratch_shapes=[
            pltpu.VMEM((ROWS, D), jnp.bfloat16),
            pltpu.VMEM((ROWS, D), jnp.bfloat16),
            pltpu.VMEM((ROWS, D), jnp.bfloat16),
            pltpu.VMEM((ROWS, D), jnp.bfloat16),
            pltpu.SemaphoreType.DMA((C,)),
            pltpu.SemaphoreType.DMA((C,)),
            pltpu.SemaphoreType.DMA((C,)),
            pltpu.SemaphoreType.DMA((C,)),
        ],
        compiler_params=pltpu.CompilerParams(collective_id=0),
    )(partial, gamma2)
